# baseline (speedup 1.0000x reference)
import functools

import jax
import jax.numpy as jnp
from jax import lax
from jax.experimental import pallas as pl
from jax.experimental.pallas import tpu as pltpu
from jax.experimental.pallas import tpu_sc as plsc

_SC_CORES = 2
_SC_SUBCORES = 16
_NW = _SC_CORES * _SC_SUBCORES
_CH = 128
_FAST_CORE = 0
_FAST_FRAC = 1.0


def _round_up(a, b):
    return ((a + b - 1) // b) * b


def _edge_loop(table, acc, src_all, dst_all, rows, sems, n):
    sg = (sems[0], sems[1])
    ss = (sems[2], sems[3])

    def rbuf(b):
        return rows.at[pl.ds(b * _CH, _CH)]

    def gather(j, b):
        return pltpu.async_copy(table.at[src_all.at[j]], rbuf(b), sg[b])

    def wait_gather(j, b):
        pltpu.make_async_copy(table.at[src_all.at[j]], rbuf(b), sg[b]).wait()

    def scatter(j, b):
        return pltpu.async_copy(rbuf(b), acc.at[dst_all.at[j]], ss[b], add=True)

    gather(0, 0)
    if n > 1:
        gather(1, 1)

    def pair(g, issue_next):
        j0 = 2 * g
        descs = []
        for b in (0, 1):
            wait_gather(j0 + b, b)
            descs.append(scatter(j0 + b, b))
        if issue_next:
            for b in (0, 1):
                descs[b].wait()
                gather(j0 + 2 + b, b)
            return None
        return descs

    if n // 2 > 1:
        lax.fori_loop(0, n // 2 - 1, lambda g, c: (pair(g, True), c)[1], 0)
    last = pair(n // 2 - 1, False)
    for d in last:
        d.wait()


def _make_segsum_flat(D, n_chunks, acc_rows):
    total_pw = n_chunks // _NW
    cpw0 = 2 * (_round_up(int(total_pw * 2 * _FAST_FRAC), 2) // 2)
    cpw0 = min(cpw0, 2 * total_pw)
    cpw1 = 2 * total_pw - cpw0
    zrows = acc_rows // _SC_SUBCORES
    mesh = plsc.VectorSubcoreMesh(core_axis_name="c", subcore_axis_name="s")

    n_out_cores = _SC_CORES if cpw1 > 0 else 1

    @functools.partial(
        pl.kernel, mesh=mesh,
        out_type=jax.ShapeDtypeStruct((n_out_cores, acc_rows, D), jnp.float32),
        compiler_params=pltpu.CompilerParams(use_tc_tiling_on_sc=False),
        scratch_types=[
            pltpu.VMEM((max(cpw0, cpw1, 1), _CH), jnp.int32),
            pltpu.VMEM((max(cpw0, cpw1, 1), _CH), jnp.int32),
            pltpu.VMEM((2 * _CH, D), jnp.float32),
            pltpu.VMEM_SHARED((acc_rows, D), jnp.float32),
            pltpu.SemaphoreType.DMA,
            pltpu.SemaphoreType.DMA,
            pltpu.SemaphoreType.DMA,
            pltpu.SemaphoreType.DMA,
        ])
    def k(table, srcm, dstm, zeros, out, src_all, dst_all, rows, acc,
          sg0, sg1, ss0, ss1):
        cid = lax.axis_index("c")
        sid = lax.axis_index("s")

        def run(cpw, base):
            pltpu.sync_copy(srcm.at[pl.ds(base, cpw)], src_all.at[pl.ds(0, cpw)])
            pltpu.sync_copy(dstm.at[pl.ds(base, cpw)], dst_all.at[pl.ds(0, cpw)])
            _edge_loop(table, acc, src_all, dst_all, rows,
                       (sg0, sg1, ss0, ss1), cpw)

        if cpw1 > 0:
            pltpu.sync_copy(zeros, acc.at[pl.ds(sid * zrows, zrows)])
            plsc.subcore_barrier()

            @pl.when(cid == _FAST_CORE)
            def _():
                run(cpw0, sid * cpw0)

            @pl.when(cid != _FAST_CORE)
            def _():
                run(cpw1, _SC_SUBCORES * cpw0 + sid * cpw1)
            plsc.subcore_barrier()
            pltpu.sync_copy(acc.at[pl.ds(sid * zrows, zrows)],
                            out.at[cid, pl.ds(sid * zrows, zrows)])
        else:
            @pl.when(cid == _FAST_CORE)
            def _():
                pltpu.sync_copy(zeros, acc.at[pl.ds(sid * zrows, zrows)])
                plsc.subcore_barrier()
                run(cpw0, sid * cpw0)
                plsc.subcore_barrier()
                pltpu.sync_copy(acc.at[pl.ds(sid * zrows, zrows)],
                                out.at[0, pl.ds(sid * zrows, zrows)])

    return k


_CFG_STRIDE = 1024


def _make_segsum_cfg(D, n_chunks):
    acc_rows = _SC_SUBCORES * _CFG_STRIDE
    mesh = plsc.VectorSubcoreMesh(core_axis_name="c", subcore_axis_name="s")

    del acc_rows

    @functools.partial(
        pl.kernel, mesh=mesh,
        out_type=jax.ShapeDtypeStruct((_NW, _CFG_STRIDE, D), jnp.float32),
        compiler_params=pltpu.CompilerParams(use_tc_tiling_on_sc=False),
        scratch_types=[
            pltpu.VMEM((n_chunks, _CH), jnp.int32),
            pltpu.VMEM((n_chunks, _CH), jnp.int32),
            pltpu.VMEM((2 * _CH, D), jnp.float32),
            pltpu.VMEM_SHARED((_SC_SUBCORES * _CFG_STRIDE, D), jnp.float32),
            pltpu.SemaphoreType.DMA,
            pltpu.SemaphoreType.DMA,
            pltpu.SemaphoreType.DMA,
            pltpu.SemaphoreType.DMA,
        ])
    def k(table, srcm, dstm, zeros, out, src_all, dst_all, rows, acc,
          sg0, sg1, ss0, ss1):
        cid = lax.axis_index("c")
        sid = lax.axis_index("s")

        @pl.when(cid == _FAST_CORE)
        def _():
            slot = acc.at[pl.ds(sid * _CFG_STRIDE, _CFG_STRIDE)]
            for rep in range(2):
                cfg = rep * _SC_SUBCORES + sid
                pltpu.sync_copy(srcm.at[cfg], src_all)
                pltpu.sync_copy(dstm.at[cfg], dst_all)
                pltpu.sync_copy(zeros, slot)
                _edge_loop(table, slot, src_all, dst_all, rows,
                           (sg0, sg1, ss0, ss1), n_chunks)
                pltpu.sync_copy(slot, out.at[cfg])

    return k


def _segsum_cfg_sc(table_3d, srcm, dstm, nc):
    c, nc_, d = table_3d.shape
    n_chunks = srcm.shape[1]
    zeros = jnp.zeros((_CFG_STRIDE, d), jnp.float32)
    k = _make_segsum_cfg(d, n_chunks)
    out = k(table_3d.reshape(c * nc_, d), srcm, dstm, zeros)
    return out[:, :nc, :]


def _pad_edges_cfg(csrc, cdst, c, nc):
    e = csrc.shape[0]
    ep = _round_up(e, 2 * _CH)
    csrc = jnp.concatenate([csrc.astype(jnp.int32), jnp.zeros((ep - e,), jnp.int32)])
    cdst = jnp.concatenate([cdst.astype(jnp.int32), jnp.full((ep - e,), nc, jnp.int32)])
    cfgs = jnp.arange(c, dtype=jnp.int32)
    srcm = (csrc[None, :] + nc * cfgs[:, None]).reshape(c, -1, _CH)
    dstm = jnp.broadcast_to(cdst[None, :], (c, cdst.shape[0])).reshape(c, -1, _CH)
    return srcm, dstm


def _segsum_big_sc(table, srcm, dstm, n, d):
    n_chunks = srcm.shape[0]
    acc_rows = _round_up(n + 1, _CH)
    zeros = jnp.zeros((acc_rows // _SC_SUBCORES, d), jnp.float32)
    k = _make_segsum_flat(d, n_chunks, acc_rows)
    p = k(table, srcm, dstm, zeros)
    return p.sum(axis=0)[:n]


def _pad_edges_flat(src, dst, n_dummy):
    e = src.shape[0]
    ep = _round_up(e, 2 * _NW * _CH)
    src = jnp.concatenate([src.astype(jnp.int32), jnp.zeros((ep - e,), jnp.int32)])
    dst = jnp.concatenate([dst.astype(jnp.int32), jnp.full((ep - e,), n_dummy, jnp.int32)])
    return src.reshape(-1, _CH), dst.reshape(-1, _CH)


def _leaky(x):
    return jax.nn.leaky_relu(x, 0.01)


def _normalize(x):
    n = jnp.linalg.norm(x, axis=-1, keepdims=True)
    return x / jnp.maximum(n, 1e-12)


def _segsum(vals_at_src, dst, n):
    return jax.ops.segment_sum(vals_at_src, dst, num_segments=n)


def _sage_pre(x, p):
    return x @ p['Wl'].T, x @ p['Wr'].T + p['bl']


def _sage_block(x, seg, deg, layers):
    def sage(xin, p):
        xl, xr = _sage_pre(xin, p)
        agg = seg(xl) / deg[:, None]
        return agg + xr
    x1 = _leaky(sage(x, layers[0]))
    x2 = sage(x1, layers[1])
    x3 = _leaky(x1 + x2)
    x4 = _leaky(sage(x3, layers[2]))
    x5 = sage(x4, layers[3])
    return _leaky(x4 + x5)


def _gat(x, s, d, n, p):
    h = x @ p['W'].T
    u = h @ p['a_src']
    v = h @ p['a_dst']
    m = jax.nn.leaky_relu(jnp.max(u) + v, 0.2)
    e = jax.nn.leaky_relu(u[s] + v[d], 0.2)
    w = jnp.exp(e - m[d])
    denom = _segsum(w[:, None], d, n)[:, 0]
    numer = _segsum(h[s] * w[:, None], d, n)
    return numer / (denom[:, None] + 1e-16) + p['b']


def kernel(node_feat, node_opcode, edge_index, node_config_feat,
           node_config_ids, config_edge_index, params):
    n = node_feat.shape[0]
    c = node_config_feat.shape[0]
    nc = node_config_ids.shape[0]
    src, dst = edge_index[0], edge_index[1]

    deg = jnp.maximum(
        jax.ops.segment_sum(jnp.ones_like(src, jnp.float32), dst, num_segments=n), 1.0)

    srcm, dstm = _pad_edges_flat(src, dst, n)
    seg_big = lambda table: _segsum_big_sc(table, srcm, dstm, n, table.shape[-1])

    x = jnp.concatenate([node_feat, params['emb'][node_opcode]], axis=1)
    x = _sage_block(x, seg_big, deg, params['model_gnn'])

    agg = seg_big(x) / deg[:, None]
    cn = _normalize(agg[node_config_ids])

    csrc, cdst = config_edge_index[0], config_edge_index[1]
    loops = jnp.arange(nc, dtype=csrc.dtype)
    s2 = jnp.concatenate([csrc, loops])
    d2 = jnp.concatenate([cdst, loops])
    g1 = _leaky(_gat(cn, s2, d2, nc, params['config_mp'][0]))
    g2 = _gat(g1, s2, d2, nc, params['config_mp'][1])
    cn = _leaky(g1 + g2)

    xs = x[node_config_ids]
    ncf = _leaky(node_config_feat @ params['prj_W'].T + params['prj_b'])
    merged = jnp.concatenate([
        jnp.broadcast_to(cn[None], (c, nc, cn.shape[-1])),
        jnp.broadcast_to(xs[None], (c, nc, xs.shape[-1])),
        ncf], axis=-1)
    merged = _normalize(merged)

    cdeg = jnp.maximum(
        jax.ops.segment_sum(jnp.ones_like(csrc, jnp.float32), cdst, num_segments=nc), 1.0)

    srcc, dstc = _pad_edges_cfg(csrc, cdst, c, nc)
    seg_cfg = lambda t: _segsum_cfg_sc(t, srcc, dstc, nc)
    hcfg = _sage_block(merged, seg_cfg, cdeg, params['config_gnn'])
    pooled = jnp.mean(hcfg, axis=1)
    h = _leaky(pooled @ params['d1'].T)
    h = _leaky(h @ params['d2'].T)
    return (h @ params['d3'].T).reshape(-1)


# speedup vs baseline: 1.1482x; 1.1482x over previous
import functools

import jax
import jax.numpy as jnp
from jax import lax
from jax.experimental import pallas as pl
from jax.experimental.pallas import tpu as pltpu
from jax.experimental.pallas import tpu_sc as plsc

_SC_CORES = 2
_SC_SUBCORES = 16
_NW = _SC_CORES * _SC_SUBCORES
_CH = 128
_FAST_CORE = 0
_FAST_FRAC = 0.5


def _round_up(a, b):
    return ((a + b - 1) // b) * b


def _edge_loop(table, acc, src_all, dst_all, rows, sems, n):
    sg = (sems[0], sems[1])
    ss = (sems[2], sems[3])

    def rbuf(b):
        return rows.at[pl.ds(b * _CH, _CH)]

    def gather(j, b):
        return pltpu.async_copy(table.at[src_all.at[j]], rbuf(b), sg[b])

    def wait_gather(j, b):
        pltpu.make_async_copy(table.at[src_all.at[j]], rbuf(b), sg[b]).wait()

    def scatter(j, b):
        return pltpu.async_copy(rbuf(b), acc.at[dst_all.at[j]], ss[b], add=True)

    gather(0, 0)
    if n > 1:
        gather(1, 1)

    def pair(g, issue_next):
        j0 = 2 * g
        descs = []
        for b in (0, 1):
            wait_gather(j0 + b, b)
            descs.append(scatter(j0 + b, b))
        if issue_next:
            for b in (0, 1):
                descs[b].wait()
                gather(j0 + 2 + b, b)
            return None
        return descs

    if n // 2 > 1:
        lax.fori_loop(0, n // 2 - 1, lambda g, c: (pair(g, True), c)[1], 0)
    last = pair(n // 2 - 1, False)
    for d in last:
        d.wait()


def _make_segsum_flat(D, n_chunks, acc_rows):
    total_pw = n_chunks // _NW
    cpw0 = 2 * (_round_up(int(total_pw * 2 * _FAST_FRAC), 2) // 2)
    cpw0 = min(cpw0, 2 * total_pw)
    cpw1 = 2 * total_pw - cpw0
    zrows = acc_rows // _SC_SUBCORES
    mesh = plsc.VectorSubcoreMesh(core_axis_name="c", subcore_axis_name="s")

    n_out_cores = _SC_CORES if cpw1 > 0 else 1

    @functools.partial(
        pl.kernel, mesh=mesh,
        out_type=jax.ShapeDtypeStruct((n_out_cores, acc_rows, D), jnp.float32),
        compiler_params=pltpu.CompilerParams(use_tc_tiling_on_sc=False),
        scratch_types=[
            pltpu.VMEM((max(cpw0, cpw1, 1), _CH), jnp.int32),
            pltpu.VMEM((max(cpw0, cpw1, 1), _CH), jnp.int32),
            pltpu.VMEM((2 * _CH, D), jnp.float32),
            pltpu.VMEM_SHARED((acc_rows, D), jnp.float32),
            pltpu.SemaphoreType.DMA,
            pltpu.SemaphoreType.DMA,
            pltpu.SemaphoreType.DMA,
            pltpu.SemaphoreType.DMA,
        ])
    def k(table, srcm, dstm, zeros, out, src_all, dst_all, rows, acc,
          sg0, sg1, ss0, ss1):
        cid = lax.axis_index("c")
        sid = lax.axis_index("s")

        def run(cpw, base):
            pltpu.sync_copy(srcm.at[pl.ds(base, cpw)], src_all.at[pl.ds(0, cpw)])
            pltpu.sync_copy(dstm.at[pl.ds(base, cpw)], dst_all.at[pl.ds(0, cpw)])
            _edge_loop(table, acc, src_all, dst_all, rows,
                       (sg0, sg1, ss0, ss1), cpw)

        if cpw1 > 0:
            pltpu.sync_copy(zeros, acc.at[pl.ds(sid * zrows, zrows)])
            plsc.subcore_barrier()

            @pl.when(cid == _FAST_CORE)
            def _():
                run(cpw0, sid * cpw0)

            @pl.when(cid != _FAST_CORE)
            def _():
                run(cpw1, _SC_SUBCORES * cpw0 + sid * cpw1)
            plsc.subcore_barrier()
            pltpu.sync_copy(acc.at[pl.ds(sid * zrows, zrows)],
                            out.at[cid, pl.ds(sid * zrows, zrows)])
        else:
            @pl.when(cid == _FAST_CORE)
            def _():
                pltpu.sync_copy(zeros, acc.at[pl.ds(sid * zrows, zrows)])
                plsc.subcore_barrier()
                run(cpw0, sid * cpw0)
                plsc.subcore_barrier()
                pltpu.sync_copy(acc.at[pl.ds(sid * zrows, zrows)],
                                out.at[0, pl.ds(sid * zrows, zrows)])

    return k


_CFG_STRIDE = 1024


def _make_segsum_cfg(D, n_chunks):
    acc_rows = _SC_SUBCORES * _CFG_STRIDE
    mesh = plsc.VectorSubcoreMesh(core_axis_name="c", subcore_axis_name="s")

    del acc_rows

    @functools.partial(
        pl.kernel, mesh=mesh,
        out_type=jax.ShapeDtypeStruct((_NW, _CFG_STRIDE, D), jnp.float32),
        compiler_params=pltpu.CompilerParams(use_tc_tiling_on_sc=False),
        scratch_types=[
            pltpu.VMEM((n_chunks, _CH), jnp.int32),
            pltpu.VMEM((n_chunks, _CH), jnp.int32),
            pltpu.VMEM((2 * _CH, D), jnp.float32),
            pltpu.VMEM_SHARED((_SC_SUBCORES * _CFG_STRIDE, D), jnp.float32),
            pltpu.SemaphoreType.DMA,
            pltpu.SemaphoreType.DMA,
            pltpu.SemaphoreType.DMA,
            pltpu.SemaphoreType.DMA,
        ])
    def k(table, srcm, dstm, zeros, out, src_all, dst_all, rows, acc,
          sg0, sg1, ss0, ss1):
        cid = lax.axis_index("c")
        sid = lax.axis_index("s")

        cfg = cid * _SC_SUBCORES + sid
        slot = acc.at[pl.ds(sid * _CFG_STRIDE, _CFG_STRIDE)]
        pltpu.sync_copy(srcm.at[cfg], src_all)
        pltpu.sync_copy(dstm.at[cfg], dst_all)
        pltpu.sync_copy(zeros, slot)
        _edge_loop(table, slot, src_all, dst_all, rows,
                   (sg0, sg1, ss0, ss1), n_chunks)
        pltpu.sync_copy(slot, out.at[cfg])

    return k


def _segsum_cfg_sc(table_3d, srcm, dstm, nc):
    c, nc_, d = table_3d.shape
    n_chunks = srcm.shape[1]
    zeros = jnp.zeros((_CFG_STRIDE, d), jnp.float32)
    k = _make_segsum_cfg(d, n_chunks)
    out = k(table_3d.reshape(c * nc_, d), srcm, dstm, zeros)
    return out[:, :nc, :]


def _pad_edges_cfg(csrc, cdst, c, nc):
    e = csrc.shape[0]
    ep = _round_up(e, 2 * _CH)
    csrc = jnp.concatenate([csrc.astype(jnp.int32), jnp.zeros((ep - e,), jnp.int32)])
    cdst = jnp.concatenate([cdst.astype(jnp.int32), jnp.full((ep - e,), nc, jnp.int32)])
    cfgs = jnp.arange(c, dtype=jnp.int32)
    srcm = (csrc[None, :] + nc * cfgs[:, None]).reshape(c, -1, _CH)
    dstm = jnp.broadcast_to(cdst[None, :], (c, cdst.shape[0])).reshape(c, -1, _CH)
    return srcm, dstm


def _segsum_big_sc(table, srcm, dstm, n, d):
    n_chunks = srcm.shape[0]
    acc_rows = _round_up(n + 1, _CH)
    zeros = jnp.zeros((acc_rows // _SC_SUBCORES, d), jnp.float32)
    k = _make_segsum_flat(d, n_chunks, acc_rows)
    p = k(table, srcm, dstm, zeros)
    return p.sum(axis=0)[:n]


def _pad_edges_flat(src, dst, n_dummy):
    e = src.shape[0]
    ep = _round_up(e, 2 * _NW * _CH)
    src = jnp.concatenate([src.astype(jnp.int32), jnp.zeros((ep - e,), jnp.int32)])
    dst = jnp.concatenate([dst.astype(jnp.int32), jnp.full((ep - e,), n_dummy, jnp.int32)])
    return src.reshape(-1, _CH), dst.reshape(-1, _CH)


def _leaky(x):
    return jax.nn.leaky_relu(x, 0.01)


def _normalize(x):
    n = jnp.linalg.norm(x, axis=-1, keepdims=True)
    return x / jnp.maximum(n, 1e-12)


def _segsum(vals_at_src, dst, n):
    return jax.ops.segment_sum(vals_at_src, dst, num_segments=n)


def _sage_pre(x, p):
    return x @ p['Wl'].T, x @ p['Wr'].T + p['bl']


def _sage_block(x, seg, deg, layers):
    def sage(xin, p):
        xl, xr = _sage_pre(xin, p)
        agg = seg(xl) / deg[:, None]
        return agg + xr
    x1 = _leaky(sage(x, layers[0]))
    x2 = sage(x1, layers[1])
    x3 = _leaky(x1 + x2)
    x4 = _leaky(sage(x3, layers[2]))
    x5 = sage(x4, layers[3])
    return _leaky(x4 + x5)


def _gat(x, s, d, n, p):
    h = x @ p['W'].T
    u = h @ p['a_src']
    v = h @ p['a_dst']
    m = jax.nn.leaky_relu(jnp.max(u) + v, 0.2)
    e = jax.nn.leaky_relu(u[s] + v[d], 0.2)
    w = jnp.exp(e - m[d])
    denom = _segsum(w[:, None], d, n)[:, 0]
    numer = _segsum(h[s] * w[:, None], d, n)
    return numer / (denom[:, None] + 1e-16) + p['b']


def kernel(node_feat, node_opcode, edge_index, node_config_feat,
           node_config_ids, config_edge_index, params):
    n = node_feat.shape[0]
    c = node_config_feat.shape[0]
    nc = node_config_ids.shape[0]
    src, dst = edge_index[0], edge_index[1]

    deg = jnp.maximum(
        jax.ops.segment_sum(jnp.ones_like(src, jnp.float32), dst, num_segments=n), 1.0)

    srcm, dstm = _pad_edges_flat(src, dst, n)
    seg_big = lambda table: _segsum_big_sc(table, srcm, dstm, n, table.shape[-1])

    x = jnp.concatenate([node_feat, params['emb'][node_opcode]], axis=1)
    x = _sage_block(x, seg_big, deg, params['model_gnn'])

    agg = seg_big(x) / deg[:, None]
    cn = _normalize(agg[node_config_ids])

    csrc, cdst = config_edge_index[0], config_edge_index[1]
    loops = jnp.arange(nc, dtype=csrc.dtype)
    s2 = jnp.concatenate([csrc, loops])
    d2 = jnp.concatenate([cdst, loops])
    g1 = _leaky(_gat(cn, s2, d2, nc, params['config_mp'][0]))
    g2 = _gat(g1, s2, d2, nc, params['config_mp'][1])
    cn = _leaky(g1 + g2)

    xs = x[node_config_ids]
    ncf = _leaky(node_config_feat @ params['prj_W'].T + params['prj_b'])
    merged = jnp.concatenate([
        jnp.broadcast_to(cn[None], (c, nc, cn.shape[-1])),
        jnp.broadcast_to(xs[None], (c, nc, xs.shape[-1])),
        ncf], axis=-1)
    merged = _normalize(merged)

    cdeg = jnp.maximum(
        jax.ops.segment_sum(jnp.ones_like(csrc, jnp.float32), cdst, num_segments=nc), 1.0)

    srcc, dstc = _pad_edges_cfg(csrc, cdst, c, nc)
    seg_cfg = lambda t: _segsum_cfg_sc(t, srcc, dstc, nc)
    hcfg = _sage_block(merged, seg_cfg, cdeg, params['config_gnn'])
    pooled = jnp.mean(hcfg, axis=1)
    h = _leaky(pooled @ params['d1'].T)
    h = _leaky(h @ params['d2'].T)
    return (h @ params['d3'].T).reshape(-1)
